# Initial kernel scaffold; baseline (speedup 1.0000x reference)
import jax
import jax.numpy as jnp
from jax.experimental import pallas as pl


def kernel(node_features, edge_index, edge_features, global_features, W_node, b_node, Wg1, bg1, Wg2, bg2, msg_W1, msg_b1, msg_W2, msg_b2, upd_W1, upd_b1, upd_W2, upd_b2, ln_g, ln_b, film_W1, film_b1, film_W2, film_b2):
    raise NotImplementedError("write your pallas kernel here")


# speedup vs baseline: 1.9649x; 1.9649x over previous
import functools

import jax
import jax.numpy as jnp
from jax import lax
from jax.experimental import pallas as pl
from jax.experimental.pallas import tpu as pltpu
from jax.experimental.pallas import tpu_sc as plsc

N = 10000
E = 160000
D = 128
DE = 16
L = 4

K = 50
NCHUNK = E // K
NW = 32
CPW = NCHUNK // NW
N_PAD = 10240
RPS = N_PAD // 16
NBLK = 1000
EBLK = 2000


def _mm_bias_body(x_ref, w_ref, b_ref, o_ref):
    o_ref[...] = (
        jnp.dot(x_ref[...], w_ref[...], preferred_element_type=jnp.float32)
        + b_ref[...]
    )


def _mm_bias(x, w, b):
    n, d = x.shape
    do = w.shape[1]
    return pl.pallas_call(
        _mm_bias_body,
        grid=(n // NBLK,),
        in_specs=[
            pl.BlockSpec((NBLK, d), lambda i: (i, 0)),
            pl.BlockSpec((d, do), lambda i: (0, 0)),
            pl.BlockSpec((1, do), lambda i: (0, 0)),
        ],
        out_specs=pl.BlockSpec((NBLK, do), lambda i: (i, 0)),
        out_shape=jax.ShapeDtypeStruct((n, do), jnp.float32),
    )(x, w, b)


def _cpre_body(ef_ref, w_ref, b_ref, o_ref):
    o_ref[0] = (
        jnp.dot(ef_ref[...], w_ref[0], preferred_element_type=jnp.float32)
        + b_ref[0]
    )


def _cpre(ef, wc, b1):
    return pl.pallas_call(
        _cpre_body,
        grid=(L, E // EBLK),
        in_specs=[
            pl.BlockSpec((EBLK, DE), lambda l, j: (j, 0)),
            pl.BlockSpec((1, DE, D), lambda l, j: (l, 0, 0)),
            pl.BlockSpec((1, 1, D), lambda l, j: (l, 0, 0)),
        ],
        out_specs=pl.BlockSpec((1, EBLK, D), lambda l, j: (l, j, 0)),
        out_shape=jax.ShapeDtypeStruct((L, E, D), jnp.float32),
    )(ef, wc, b1)


def _pre_body(h_ref, wa_ref, wb_ref, a_ref, b_ref):
    h = h_ref[...]
    a_ref[...] = jnp.dot(h, wa_ref[...], preferred_element_type=jnp.float32)
    b_ref[...] = jnp.dot(h, wb_ref[...], preferred_element_type=jnp.float32)


def _pre(h, wa, wb):
    return pl.pallas_call(
        _pre_body,
        grid=(N // NBLK,),
        in_specs=[
            pl.BlockSpec((NBLK, D), lambda i: (i, 0)),
            pl.BlockSpec((D, D), lambda i: (0, 0)),
            pl.BlockSpec((D, D), lambda i: (0, 0)),
        ],
        out_specs=[
            pl.BlockSpec((NBLK, D), lambda i: (i, 0)),
            pl.BlockSpec((NBLK, D), lambda i: (i, 0)),
        ],
        out_shape=[
            jax.ShapeDtypeStruct((N, D), jnp.float32),
            jax.ShapeDtypeStruct((N, D), jnp.float32),
        ],
    )(h, wa, wb)


def _post_body(h_ref, sa_ref, sb_ref, da_ref, db_ref, w2_ref, b2_ref,
               u1a_ref, u1b_ref, ub1_ref, u2_ref, ub2_ref, lng_ref, lnb_ref,
               gam_ref, bet_ref, o_ref):
    h = h_ref[...]
    s = sa_ref[0] + sb_ref[0]
    deg = da_ref[0] + db_ref[0]
    agg = (
        jnp.dot(s, w2_ref[...], preferred_element_type=jnp.float32)
        + deg * b2_ref[...]
    )
    u = (
        jnp.dot(h, u1a_ref[...], preferred_element_type=jnp.float32)
        + jnp.dot(agg, u1b_ref[...], preferred_element_type=jnp.float32)
        + ub1_ref[...]
    )
    su = u / (1.0 + jnp.exp(-u))
    t = jnp.dot(su, u2_ref[...], preferred_element_type=jnp.float32) + ub2_ref[...]
    x = h + t
    mu = jnp.mean(x, axis=1, keepdims=True)
    xc = x - mu
    var = jnp.mean(xc * xc, axis=1, keepdims=True)
    hln = xc / jnp.sqrt(var + 1e-5) * lng_ref[...] + lnb_ref[...]
    o_ref[...] = hln * (1.0 + gam_ref[...]) + bet_ref[...]


def _post(h, s_part, deg_part, w2, b2, u1a, u1b, ub1, u2, ub2, lng, lnb,
          gam, bet):
    full = lambda shape, idx: pl.BlockSpec(shape, lambda i: idx)
    return pl.pallas_call(
        _post_body,
        grid=(N // NBLK,),
        in_specs=[
            pl.BlockSpec((NBLK, D), lambda i: (i, 0)),
            pl.BlockSpec((1, NBLK, D), lambda i: (0, i, 0)),
            pl.BlockSpec((1, NBLK, D), lambda i: (1, i, 0)),
            pl.BlockSpec((1, NBLK, D), lambda i: (0, i, 0)),
            pl.BlockSpec((1, NBLK, D), lambda i: (1, i, 0)),
            full((D, D), (0, 0)),
            full((1, D), (0, 0)),
            full((D, D), (0, 0)),
            full((D, D), (0, 0)),
            full((1, D), (0, 0)),
            full((D, D), (0, 0)),
            full((1, D), (0, 0)),
            full((1, D), (0, 0)),
            full((1, D), (0, 0)),
            full((1, D), (0, 0)),
            full((1, D), (0, 0)),
        ],
        out_specs=pl.BlockSpec((NBLK, D), lambda i: (i, 0)),
        out_shape=jax.ShapeDtypeStruct((N, D), jnp.float32),
    )(h, s_part, s_part, deg_part, deg_part, w2, b2, u1a, u1b, ub1, u2, ub2,
      lng, lnb, gam, bet)


def _sc_agg(a_nd, b_nd, c_l, src2d, dst2d, zeros_aug):
    mesh = plsc.VectorSubcoreMesh(core_axis_name="c", subcore_axis_name="s")

    @functools.partial(
        pl.kernel,
        out_type=jax.ShapeDtypeStruct((2, N_PAD, D), jnp.float32),
        mesh=mesh,
        scratch_types=[
            pltpu.VMEM((CPW, K), jnp.int32),
            pltpu.VMEM((CPW, K), jnp.int32),
            pltpu.VMEM((K, D), jnp.float32),
            pltpu.VMEM((K, D), jnp.float32),
            pltpu.VMEM((K, D), jnp.float32),
            pltpu.VMEM_SHARED((N_PAD, D), jnp.float32),
            pltpu.SemaphoreType.DMA,
            pltpu.SemaphoreType.DMA,
        ],
    )
    def k(a_hbm, b_hbm, c_hbm, src_hbm, dst_hbm, z_hbm, out_hbm,
          src_v, dst_v, a_v, b_v, c_v, s_sh, sem_a, sem_b):
        cid = lax.axis_index("c")
        sid = lax.axis_index("s")
        wid = sid * 2 + cid
        base = wid * CPW
        pltpu.sync_copy(src_hbm.at[wid], src_v)
        pltpu.sync_copy(dst_hbm.at[wid], dst_v)
        r0 = sid * RPS
        pltpu.sync_copy(z_hbm.at[pl.ds(r0, RPS)], s_sh.at[pl.ds(r0, RPS)])
        plsc.subcore_barrier()

        def chunk(j, carry):
            ca = pltpu.async_copy(a_hbm.at[src_v.at[j]], a_v, sem_a)
            cb = pltpu.async_copy(b_hbm.at[dst_v.at[j]], b_v, sem_b)
            pltpu.sync_copy(c_hbm.at[base + j], c_v)
            ca.wait()
            cb.wait()

            def row(r, c2):
                for i in range(D // 16):
                    sl = pl.ds(i * 16, 16)
                    z = a_v[r, sl] + b_v[r, sl] + c_v[r, sl]
                    a_v[r, sl] = z / (1.0 + jnp.exp(-z))
                return c2

            lax.fori_loop(0, K, row, 0)
            pltpu.sync_copy(a_v, s_sh.at[dst_v.at[j]], add=True)
            return carry

        lax.fori_loop(0, CPW, chunk, 0)
        plsc.subcore_barrier()
        pltpu.sync_copy(s_sh.at[pl.ds(r0, RPS)],
                        out_hbm.at[cid, pl.ds(r0, RPS)])

    return k(a_nd, b_nd, c_l, src2d, dst2d, zeros_aug)


def _sc_deg(dst3d, zeros128):
    mesh = plsc.VectorSubcoreMesh(core_axis_name="c", subcore_axis_name="s")

    @functools.partial(
        pl.kernel,
        out_type=jax.ShapeDtypeStruct((2, N_PAD, D), jnp.float32),
        mesh=mesh,
        scratch_types=[
            pltpu.VMEM((CPW, K), jnp.int32),
            pltpu.VMEM((K, D), jnp.float32),
            pltpu.VMEM_SHARED((N_PAD, D), jnp.float32),
        ],
    )
    def k(dst_hbm, z_hbm, out_hbm, dst_v, ones_v, s_sh):
        cid = lax.axis_index("c")
        sid = lax.axis_index("s")
        wid = sid * 2 + cid
        pltpu.sync_copy(dst_hbm.at[wid], dst_v)
        r0 = sid * RPS
        pltpu.sync_copy(z_hbm.at[pl.ds(r0, RPS)], s_sh.at[pl.ds(r0, RPS)])

        one = jnp.full((16,), 1.0, jnp.float32)

        def fill(r, c2):
            for i in range(D // 16):
                ones_v[r, pl.ds(i * 16, 16)] = one
            return c2

        lax.fori_loop(0, K, fill, 0)
        plsc.subcore_barrier()

        def chunk(j, carry):
            pltpu.sync_copy(ones_v, s_sh.at[dst_v.at[j]], add=True)
            return carry

        lax.fori_loop(0, CPW, chunk, 0)
        plsc.subcore_barrier()
        pltpu.sync_copy(s_sh.at[pl.ds(r0, RPS)],
                        out_hbm.at[cid, pl.ds(r0, RPS)])

    return k(dst3d, zeros128)


def kernel(node_features, edge_index, edge_features, global_features, W_node,
           b_node, Wg1, bg1, Wg2, bg2, msg_W1, msg_b1, msg_W2, msg_b2, upd_W1,
           upd_b1, upd_W2, upd_b2, ln_g, ln_b, film_W1, film_b1, film_W2,
           film_b2):
    nf = node_features[0]
    src3d = edge_index[0].reshape(NW, CPW, K)
    dst3d = edge_index[1].reshape(NW, CPW, K)

    g = global_features @ Wg1 + bg1
    g = (g / (1.0 + jnp.exp(-g))) @ Wg2 + bg2
    gammas, betas = [], []
    for l in range(L):
        f = g @ film_W1[l] + film_b1[l]
        f = (f / (1.0 + jnp.exp(-f))) @ film_W2[l] + film_b2[l]
        gammas.append(f[:, :D])
        betas.append(f[:, D:])

    h = _mm_bias(nf, W_node, b_node.reshape(1, D))

    c_all = _cpre(edge_features, msg_W1[:, 2 * D:, :],
                  msg_b1.reshape(L, 1, D))
    c_all = c_all.reshape(L, NCHUNK, K, D)

    zeros128 = jnp.zeros((N_PAD, D), jnp.float32)
    deg_part = _sc_deg(dst3d, zeros128)

    for l in range(L):
        a_nd, b_nd = _pre(h, msg_W1[l, :D, :], msg_W1[l, D:2 * D, :])
        s_part = _sc_agg(a_nd, b_nd, c_all[l], src3d, dst3d, zeros128)
        h = _post(h, s_part, deg_part, msg_W2[l], msg_b2[l].reshape(1, D),
                  upd_W1[l, :D, :], upd_W1[l, D:, :], upd_b1[l].reshape(1, D),
                  upd_W2[l], upd_b2[l].reshape(1, D), ln_g[l].reshape(1, D),
                  ln_b[l].reshape(1, D), gammas[l], betas[l])

    return h[None]


# speedup vs baseline: 2.0219x; 1.0290x over previous
import functools

import jax
import jax.numpy as jnp
from jax import lax
from jax.experimental import pallas as pl
from jax.experimental.pallas import tpu as pltpu
from jax.experimental.pallas import tpu_sc as plsc

N = 10000
E = 160000
D = 128
DE = 16
L = 4

K = 50
NCHUNK = E // K
NW = 32
CPW = NCHUNK // NW
N_PAD = 10240
RPS = N_PAD // 16
NBLK = 1000
EBLK = 2000


def _mm_bias_body(x_ref, w_ref, b_ref, o_ref):
    o_ref[...] = (
        jnp.dot(x_ref[...], w_ref[...], preferred_element_type=jnp.float32)
        + b_ref[...]
    )


def _mm_bias(x, w, b):
    n, d = x.shape
    do = w.shape[1]
    return pl.pallas_call(
        _mm_bias_body,
        grid=(n // NBLK,),
        in_specs=[
            pl.BlockSpec((NBLK, d), lambda i: (i, 0)),
            pl.BlockSpec((d, do), lambda i: (0, 0)),
            pl.BlockSpec((1, do), lambda i: (0, 0)),
        ],
        out_specs=pl.BlockSpec((NBLK, do), lambda i: (i, 0)),
        out_shape=jax.ShapeDtypeStruct((n, do), jnp.float32),
    )(x, w, b)


def _cpre_body(ef_ref, w_ref, b_ref, o_ref):
    o_ref[0] = (
        jnp.dot(ef_ref[...], w_ref[0], preferred_element_type=jnp.float32)
        + b_ref[0]
    )


def _cpre(ef, wc, b1):
    return pl.pallas_call(
        _cpre_body,
        grid=(L, E // EBLK),
        in_specs=[
            pl.BlockSpec((EBLK, DE), lambda l, j: (j, 0)),
            pl.BlockSpec((1, DE, D), lambda l, j: (l, 0, 0)),
            pl.BlockSpec((1, 1, D), lambda l, j: (l, 0, 0)),
        ],
        out_specs=pl.BlockSpec((1, EBLK, D), lambda l, j: (l, j, 0)),
        out_shape=jax.ShapeDtypeStruct((L, E, D), jnp.float32),
    )(ef, wc, b1)


def _pre_body(h_ref, wa_ref, wb_ref, a_ref, b_ref):
    h = h_ref[...]
    a_ref[...] = jnp.dot(h, wa_ref[...], preferred_element_type=jnp.float32)
    b_ref[...] = jnp.dot(h, wb_ref[...], preferred_element_type=jnp.float32)


def _pre(h, wa, wb):
    return pl.pallas_call(
        _pre_body,
        grid=(N // NBLK,),
        in_specs=[
            pl.BlockSpec((NBLK, D), lambda i: (i, 0)),
            pl.BlockSpec((D, D), lambda i: (0, 0)),
            pl.BlockSpec((D, D), lambda i: (0, 0)),
        ],
        out_specs=[
            pl.BlockSpec((NBLK, D), lambda i: (i, 0)),
            pl.BlockSpec((NBLK, D), lambda i: (i, 0)),
        ],
        out_shape=[
            jax.ShapeDtypeStruct((N, D), jnp.float32),
            jax.ShapeDtypeStruct((N, D), jnp.float32),
        ],
    )(h, wa, wb)


def _post_body(h_ref, sa_ref, sb_ref, da_ref, db_ref, w2_ref, b2_ref,
               u1a_ref, u1b_ref, ub1_ref, u2_ref, ub2_ref, lng_ref, lnb_ref,
               gam_ref, bet_ref, o_ref):
    h = h_ref[...]
    s = sa_ref[0] + sb_ref[0]
    deg = da_ref[0] + db_ref[0]
    agg = (
        jnp.dot(s, w2_ref[...], preferred_element_type=jnp.float32)
        + deg * b2_ref[...]
    )
    u = (
        jnp.dot(h, u1a_ref[...], preferred_element_type=jnp.float32)
        + jnp.dot(agg, u1b_ref[...], preferred_element_type=jnp.float32)
        + ub1_ref[...]
    )
    su = u / (1.0 + jnp.exp(-u))
    t = jnp.dot(su, u2_ref[...], preferred_element_type=jnp.float32) + ub2_ref[...]
    x = h + t
    mu = jnp.mean(x, axis=1, keepdims=True)
    xc = x - mu
    var = jnp.mean(xc * xc, axis=1, keepdims=True)
    hln = xc / jnp.sqrt(var + 1e-5) * lng_ref[...] + lnb_ref[...]
    o_ref[...] = hln * (1.0 + gam_ref[...]) + bet_ref[...]


def _post(h, s_part, deg_part, w2, b2, u1a, u1b, ub1, u2, ub2, lng, lnb,
          gam, bet):
    full = lambda shape, idx: pl.BlockSpec(shape, lambda i: idx)
    return pl.pallas_call(
        _post_body,
        grid=(N // NBLK,),
        in_specs=[
            pl.BlockSpec((NBLK, D), lambda i: (i, 0)),
            pl.BlockSpec((1, NBLK, D), lambda i: (0, i, 0)),
            pl.BlockSpec((1, NBLK, D), lambda i: (1, i, 0)),
            pl.BlockSpec((1, NBLK, D), lambda i: (0, i, 0)),
            pl.BlockSpec((1, NBLK, D), lambda i: (1, i, 0)),
            full((D, D), (0, 0)),
            full((1, D), (0, 0)),
            full((D, D), (0, 0)),
            full((D, D), (0, 0)),
            full((1, D), (0, 0)),
            full((D, D), (0, 0)),
            full((1, D), (0, 0)),
            full((1, D), (0, 0)),
            full((1, D), (0, 0)),
            full((1, D), (0, 0)),
            full((1, D), (0, 0)),
        ],
        out_specs=pl.BlockSpec((NBLK, D), lambda i: (i, 0)),
        out_shape=jax.ShapeDtypeStruct((N, D), jnp.float32),
    )(h, s_part, s_part, deg_part, deg_part, w2, b2, u1a, u1b, ub1, u2, ub2,
      lng, lnb, gam, bet)


def _sc_agg(a_nd, b_nd, c_l, src2d, dst2d, zeros_aug):
    mesh = plsc.VectorSubcoreMesh(core_axis_name="c", subcore_axis_name="s")

    @functools.partial(
        pl.kernel,
        out_type=jax.ShapeDtypeStruct((2, N_PAD, D), jnp.float32),
        mesh=mesh,
        scratch_types=[
            pltpu.VMEM((CPW, K), jnp.int32),
            pltpu.VMEM((CPW, K), jnp.int32),
            pltpu.VMEM((K, D), jnp.float32),
            pltpu.VMEM((K, D), jnp.float32),
            pltpu.VMEM((K, D), jnp.float32),
            pltpu.VMEM_SHARED((N_PAD, D), jnp.float32),
            pltpu.SemaphoreType.DMA,
            pltpu.SemaphoreType.DMA,
        ],
    )
    def k(a_hbm, b_hbm, c_hbm, src_hbm, dst_hbm, z_hbm, out_hbm,
          src_v, dst_v, a_v, b_v, c_v, s_sh, sem_a, sem_b):
        cid = lax.axis_index("c")
        sid = lax.axis_index("s")
        wid = sid * 2 + cid
        base = wid * CPW
        pltpu.sync_copy(src_hbm.at[wid], src_v)
        pltpu.sync_copy(dst_hbm.at[wid], dst_v)
        r0 = sid * RPS
        pltpu.sync_copy(z_hbm.at[pl.ds(r0, RPS)], s_sh.at[pl.ds(r0, RPS)])
        plsc.subcore_barrier()

        def chunk(j, carry):
            ca = pltpu.async_copy(a_hbm.at[src_v.at[j]], a_v, sem_a)
            cb = pltpu.async_copy(b_hbm.at[dst_v.at[j]], b_v, sem_b)
            pltpu.sync_copy(c_hbm.at[base + j], c_v)
            ca.wait()
            cb.wait()

            @plsc.parallel_loop(0, K, unroll=2)
            def row(r):
                for i in range(D // 16):
                    sl = pl.ds(i * 16, 16)
                    z = a_v[r, sl] + b_v[r, sl] + c_v[r, sl]
                    a_v[r, sl] = z / (1.0 + jnp.exp(-z))
            pltpu.sync_copy(a_v, s_sh.at[dst_v.at[j]], add=True)
            return carry

        lax.fori_loop(0, CPW, chunk, 0)
        plsc.subcore_barrier()
        pltpu.sync_copy(s_sh.at[pl.ds(r0, RPS)],
                        out_hbm.at[cid, pl.ds(r0, RPS)])

    return k(a_nd, b_nd, c_l, src2d, dst2d, zeros_aug)


def _sc_deg(dst3d, zeros128):
    mesh = plsc.VectorSubcoreMesh(core_axis_name="c", subcore_axis_name="s")

    @functools.partial(
        pl.kernel,
        out_type=jax.ShapeDtypeStruct((2, N_PAD, D), jnp.float32),
        mesh=mesh,
        scratch_types=[
            pltpu.VMEM((CPW, K), jnp.int32),
            pltpu.VMEM((K, D), jnp.float32),
            pltpu.VMEM_SHARED((N_PAD, D), jnp.float32),
        ],
    )
    def k(dst_hbm, z_hbm, out_hbm, dst_v, ones_v, s_sh):
        cid = lax.axis_index("c")
        sid = lax.axis_index("s")
        wid = sid * 2 + cid
        pltpu.sync_copy(dst_hbm.at[wid], dst_v)
        r0 = sid * RPS
        pltpu.sync_copy(z_hbm.at[pl.ds(r0, RPS)], s_sh.at[pl.ds(r0, RPS)])

        one = jnp.full((16,), 1.0, jnp.float32)

        @plsc.parallel_loop(0, K, unroll=2)
        def fill(r):
            for i in range(D // 16):
                ones_v[r, pl.ds(i * 16, 16)] = one
        plsc.subcore_barrier()

        def chunk(j, carry):
            pltpu.sync_copy(ones_v, s_sh.at[dst_v.at[j]], add=True)
            return carry

        lax.fori_loop(0, CPW, chunk, 0)
        plsc.subcore_barrier()
        pltpu.sync_copy(s_sh.at[pl.ds(r0, RPS)],
                        out_hbm.at[cid, pl.ds(r0, RPS)])

    return k(dst3d, zeros128)


def kernel(node_features, edge_index, edge_features, global_features, W_node,
           b_node, Wg1, bg1, Wg2, bg2, msg_W1, msg_b1, msg_W2, msg_b2, upd_W1,
           upd_b1, upd_W2, upd_b2, ln_g, ln_b, film_W1, film_b1, film_W2,
           film_b2):
    nf = node_features[0]
    src3d = edge_index[0].reshape(NW, CPW, K)
    dst3d = edge_index[1].reshape(NW, CPW, K)

    g = global_features @ Wg1 + bg1
    g = (g / (1.0 + jnp.exp(-g))) @ Wg2 + bg2
    gammas, betas = [], []
    for l in range(L):
        f = g @ film_W1[l] + film_b1[l]
        f = (f / (1.0 + jnp.exp(-f))) @ film_W2[l] + film_b2[l]
        gammas.append(f[:, :D])
        betas.append(f[:, D:])

    h = _mm_bias(nf, W_node, b_node.reshape(1, D))

    c_all = _cpre(edge_features, msg_W1[:, 2 * D:, :],
                  msg_b1.reshape(L, 1, D))
    c_all = c_all.reshape(L, NCHUNK, K, D)

    zeros128 = jnp.zeros((N_PAD, D), jnp.float32)
    deg_part = _sc_deg(dst3d, zeros128)

    for l in range(L):
        a_nd, b_nd = _pre(h, msg_W1[l, :D, :], msg_W1[l, D:2 * D, :])
        s_part = _sc_agg(a_nd, b_nd, c_all[l], src3d, dst3d, zeros128)
        h = _post(h, s_part, deg_part, msg_W2[l], msg_b2[l].reshape(1, D),
                  upd_W1[l, :D, :], upd_W1[l, D:, :], upd_b1[l].reshape(1, D),
                  upd_W2[l], upd_b2[l].reshape(1, D), ln_g[l].reshape(1, D),
                  ln_b[l].reshape(1, D), gammas[l], betas[l])

    return h[None]
